# baseline (speedup 1.0000x reference)
import jax
import jax.numpy as jnp
import numpy as np
from jax.experimental import pallas as pl
from jax.sharding import Mesh, PartitionSpec as P

B, N, C, H = 2, 2048, 768, 12
D = C // H
HID = 4 * C
EPS = 1e-5
SCALE = D ** -0.5

RBLK = 1024

_QSCALE = np.concatenate([np.full((C,), SCALE, np.float32),
                          np.ones((2 * C,), np.float32)])


def _layernorm(xf, g, b):
    mu = jnp.mean(xf, axis=-1, keepdims=True)
    xc = xf - mu
    var = jnp.mean(xc * xc, axis=-1, keepdims=True)
    return xc * jax.lax.rsqrt(var + EPS) * g + b


def _qkv_kernel(x_ref, g_ref, b_ref, w_ref, bias_ref, out_ref):
    for sub in range(2):
        rows = pl.ds(sub * (RBLK // 2), RBLK // 2)
        h = _layernorm(x_ref[rows, :], g_ref[...],
                       b_ref[...]).astype(jnp.bfloat16)
        acc = jax.lax.dot_general(
            h, w_ref[...], (((1,), (0,)), ((), ())),
            preferred_element_type=jnp.float32)
        out_ref[rows, :] = (acc + bias_ref[...]).astype(jnp.bfloat16)


def _attn_kernel(q_ref, k_ref, v_ref, o_ref):
    outs = []
    for j in range(2):
        sl = pl.ds(j * D, D)
        q = q_ref[:, sl]
        k = k_ref[:, sl]
        v = v_ref[:, sl]
        s = jax.lax.dot_general(
            q, k, (((1,), (1,)), ((), ())),
            preferred_element_type=jnp.float32)
        p = jnp.exp(s.astype(jnp.bfloat16))
        v_aug = jnp.concatenate(
            [v, jnp.ones((N, D), jnp.bfloat16)], axis=1)
        o_aug = jax.lax.dot_general(
            p, v_aug, (((1,), (0,)), ((), ())),
            preferred_element_type=jnp.float32)
        outs.append((o_aug[:, :D] / o_aug[:, D:D + 1]).astype(jnp.bfloat16))
    o_ref[...] = jnp.concatenate(outs, axis=1)


def _mlp_kernel(o_ref, x_ref, pw_ref, pb_ref, g2_ref, b2_ref,
                w1_ref, b1_ref, w2_ref, b2b_ref, out_ref):
    for sub in range(2):
        rows = pl.ds(sub * (RBLK // 2), RBLK // 2)
        proj = jax.lax.dot_general(
            o_ref[rows, :], pw_ref[...], (((1,), (0,)), ((), ())),
            preferred_element_type=jnp.float32)
        x1 = proj + pb_ref[...] + x_ref[rows, :]
        h = _layernorm(x1, g2_ref[...], b2_ref[...]).astype(jnp.bfloat16)
        h1 = jax.lax.dot_general(
            h, w1_ref[...], (((1,), (0,)), ((), ())),
            preferred_element_type=jnp.float32) + b1_ref[...]
        g = 0.5 * h1 * (1.0 + jax.lax.erf(h1 * (2.0 ** -0.5)))
        h2 = jax.lax.dot_general(
            g.astype(jnp.bfloat16), w2_ref[...], (((1,), (0,)), ((), ())),
            preferred_element_type=jnp.float32)
        out_ref[rows, :] = h2 + b2b_ref[...] + x1


def _block(x, norm1_g, norm1_b, qkv_wc, qkv_bc, proj_w, proj_b,
           norm2_g, norm2_b, fc1_w, fc1_b, fc2_w, fc2_b):
    bl = x.shape[0]
    rows = bl * N
    nq = N // RBLK
    xf = x.reshape(rows, C)
    row2 = lambda a: a.reshape(1, -1)

    qkv = pl.pallas_call(
        _qkv_kernel,
        grid=(rows // RBLK,),
        in_specs=[
            pl.BlockSpec((RBLK, C), lambda i: (i, 0)),
            pl.BlockSpec((1, C), lambda i: (0, 0)),
            pl.BlockSpec((1, C), lambda i: (0, 0)),
            pl.BlockSpec((C, 3 * C), lambda i: (0, 0)),
            pl.BlockSpec((1, 3 * C), lambda i: (0, 0)),
        ],
        out_specs=pl.BlockSpec((RBLK, 3 * C), lambda i: (i, 0)),
        out_shape=jax.ShapeDtypeStruct((rows, 3 * C), jnp.bfloat16),
    )(xf, row2(norm1_g), row2(norm1_b), qkv_wc, row2(qkv_bc))

    attn_out = pl.pallas_call(
        _attn_kernel,
        grid=(bl, H // 2, nq),
        in_specs=[
            pl.BlockSpec((RBLK, 2 * D), lambda b, h, i: (b * nq + i, h)),
            pl.BlockSpec((N, 2 * D), lambda b, h, i: (b, H // 2 + h)),
            pl.BlockSpec((N, 2 * D), lambda b, h, i: (b, H + h)),
        ],
        out_specs=pl.BlockSpec((RBLK, 2 * D), lambda b, h, i: (b * nq + i, h)),
        out_shape=jax.ShapeDtypeStruct((rows, C), jnp.bfloat16),
    )(qkv, qkv, qkv)

    out = pl.pallas_call(
        _mlp_kernel,
        grid=(rows // RBLK,),
        in_specs=[
            pl.BlockSpec((RBLK, C), lambda i: (i, 0)),
            pl.BlockSpec((RBLK, C), lambda i: (i, 0)),
            pl.BlockSpec((C, C), lambda i: (0, 0)),
            pl.BlockSpec((1, C), lambda i: (0, 0)),
            pl.BlockSpec((1, C), lambda i: (0, 0)),
            pl.BlockSpec((1, C), lambda i: (0, 0)),
            pl.BlockSpec((C, HID), lambda i: (0, 0)),
            pl.BlockSpec((1, HID), lambda i: (0, 0)),
            pl.BlockSpec((HID, C), lambda i: (0, 0)),
            pl.BlockSpec((1, C), lambda i: (0, 0)),
        ],
        out_specs=pl.BlockSpec((RBLK, C), lambda i: (i, 0)),
        out_shape=jax.ShapeDtypeStruct((rows, C), jnp.float32),
    )(attn_out, xf, proj_w.astype(jnp.bfloat16), row2(proj_b),
      row2(norm2_g), row2(norm2_b),
      fc1_w.astype(jnp.bfloat16), row2(fc1_b),
      fc2_w.astype(jnp.bfloat16), row2(fc2_b))

    return out.reshape(bl, N, C)


@jax.jit
def kernel(x, norm1_g, norm1_b, qkv_w, qkv_b, proj_w, proj_b,
           norm2_g, norm2_b, fc1_w, fc1_b, fc2_w, fc2_b):
    qkv_wc = (qkv_w * _QSCALE).astype(jnp.bfloat16)
    qkv_bc = qkv_b * _QSCALE
    args = (norm1_g, norm1_b, qkv_wc, qkv_bc, proj_w, proj_b,
            norm2_g, norm2_b, fc1_w, fc1_b, fc2_w, fc2_b)

    ndev = jax.device_count()
    if ndev >= 2 and B % 2 == 0:
        mesh = Mesh(np.array(jax.devices()[:2]), ("b",))
        f = jax.shard_map(
            _block,
            mesh=mesh,
            in_specs=(P("b"),) + (P(),) * len(args),
            out_specs=P("b"),
            check_vma=False,
        )
        return f(x, *args)
    return _block(x, *args)


# speedup vs baseline: 2.9389x; 2.9389x over previous
import jax
import jax.numpy as jnp
import numpy as np
from jax.experimental import pallas as pl
from jax.experimental.pallas import tpu as pltpu

B, N, C, H = 2, 2048, 768, 12
D = C // H
HID = 4 * C
EPS = 1e-5
SCALE = D ** -0.5

ROWS = B * N
RBLK = 1024
NQ = N // RBLK
H2 = H // 2

_QSCALE = np.concatenate([np.full((C,), SCALE, np.float32),
                          np.ones((2 * C,), np.float32)])


def _layernorm(xf, g, b):
    mu = jnp.mean(xf, axis=-1, keepdims=True)
    xc = xf - mu
    var = jnp.mean(xc * xc, axis=-1, keepdims=True)
    return xc * jax.lax.rsqrt(var + EPS) * g + b


def _qkv_kernel(x_ref, g_ref, b_ref, w_ref, bias_ref, out_ref):
    for sub in range(2):
        rows = pl.ds(sub * (RBLK // 2), RBLK // 2)
        h = _layernorm(x_ref[rows, :], g_ref[...],
                       b_ref[...]).astype(jnp.bfloat16)
        acc = jax.lax.dot_general(
            h, w_ref[...], (((1,), (0,)), ((), ())),
            preferred_element_type=jnp.float32)
        out_ref[rows, :] = (acc + bias_ref[...]).astype(jnp.bfloat16)


def _attn_mlp_kernel(q_ref, k_ref, v_ref, x_ref, pw_ref, pb_ref,
                     g2_ref, b2_ref, w1_ref, b1_ref, w2_ref, b2b_ref,
                     out_ref, acc_ref):
    hp = pl.program_id(2)

    outs = []
    for j in range(2):
        sl = pl.ds(j * D, D)
        q = q_ref[:, sl]
        k = k_ref[:, sl]
        v = v_ref[:, sl]
        s = jax.lax.dot_general(
            q, k, (((1,), (1,)), ((), ())),
            preferred_element_type=jnp.float32)
        p = jnp.exp(s.astype(jnp.bfloat16))
        v_aug = jnp.concatenate(
            [v, jnp.ones((N, D), jnp.bfloat16)], axis=1)
        o_aug = jax.lax.dot_general(
            p, v_aug, (((1,), (0,)), ((), ())),
            preferred_element_type=jnp.float32)
        outs.append((o_aug[:, :D] / o_aug[:, D:D + 1]).astype(jnp.bfloat16))
    o2 = jnp.concatenate(outs, axis=1)

    part = jax.lax.dot_general(
        o2, pw_ref[...], (((1,), (0,)), ((), ())),
        preferred_element_type=jnp.float32)

    @pl.when(hp == 0)
    def _():
        acc_ref[...] = x_ref[...] + pb_ref[...] + part

    @pl.when(hp != 0)
    def _():
        acc_ref[...] += part

    @pl.when(hp == H2 - 1)
    def _():
        for sub in range(2):
            rows = pl.ds(sub * (RBLK // 2), RBLK // 2)
            x1 = acc_ref[rows, :]
            h = _layernorm(x1, g2_ref[...], b2_ref[...]).astype(jnp.bfloat16)
            h1 = jax.lax.dot_general(
                h, w1_ref[...], (((1,), (0,)), ((), ())),
                preferred_element_type=jnp.float32) + b1_ref[...]
            g = 0.5 * h1 * (1.0 + jax.lax.erf(h1 * (2.0 ** -0.5)))
            h2 = jax.lax.dot_general(
                g.astype(jnp.bfloat16), w2_ref[...], (((1,), (0,)), ((), ())),
                preferred_element_type=jnp.float32)
            out_ref[rows, :] = h2 + b2b_ref[...] + x1


@jax.jit
def kernel(x, norm1_g, norm1_b, qkv_w, qkv_b, proj_w, proj_b,
           norm2_g, norm2_b, fc1_w, fc1_b, fc2_w, fc2_b):
    xf = x.reshape(ROWS, C)
    row2 = lambda a: a.reshape(1, -1)

    qkv = pl.pallas_call(
        _qkv_kernel,
        grid=(ROWS // RBLK,),
        in_specs=[
            pl.BlockSpec((RBLK, C), lambda i: (i, 0)),
            pl.BlockSpec((1, C), lambda i: (0, 0)),
            pl.BlockSpec((1, C), lambda i: (0, 0)),
            pl.BlockSpec((C, 3 * C), lambda i: (0, 0)),
            pl.BlockSpec((1, 3 * C), lambda i: (0, 0)),
        ],
        out_specs=pl.BlockSpec((RBLK, 3 * C), lambda i: (i, 0)),
        out_shape=jax.ShapeDtypeStruct((ROWS, 3 * C), jnp.bfloat16),
    )(xf, row2(norm1_g), row2(norm1_b),
      (qkv_w * _QSCALE).astype(jnp.bfloat16), row2(qkv_b * _QSCALE))

    out = pl.pallas_call(
        _attn_mlp_kernel,
        grid=(B, NQ, H2),
        in_specs=[
            pl.BlockSpec((RBLK, 2 * D), lambda b, i, h: (b * NQ + i, h)),
            pl.BlockSpec((N, 2 * D), lambda b, i, h: (b, H2 + h)),
            pl.BlockSpec((N, 2 * D), lambda b, i, h: (b, H + h)),
            pl.BlockSpec((RBLK, C), lambda b, i, h: (b * NQ + i, 0)),
            pl.BlockSpec((2 * D, C), lambda b, i, h: (h, 0)),
            pl.BlockSpec((1, C), lambda b, i, h: (0, 0)),
            pl.BlockSpec((1, C), lambda b, i, h: (0, 0)),
            pl.BlockSpec((1, C), lambda b, i, h: (0, 0)),
            pl.BlockSpec((C, HID), lambda b, i, h: (0, 0)),
            pl.BlockSpec((1, HID), lambda b, i, h: (0, 0)),
            pl.BlockSpec((HID, C), lambda b, i, h: (0, 0)),
            pl.BlockSpec((1, C), lambda b, i, h: (0, 0)),
        ],
        out_specs=pl.BlockSpec((RBLK, C), lambda b, i, h: (b * NQ + i, 0)),
        out_shape=jax.ShapeDtypeStruct((ROWS, C), jnp.float32),
        scratch_shapes=[pltpu.VMEM((RBLK, C), jnp.float32)],
    )(qkv, qkv, qkv, xf,
      proj_w.astype(jnp.bfloat16), row2(proj_b),
      row2(norm2_g), row2(norm2_b),
      fc1_w.astype(jnp.bfloat16), row2(fc1_b),
      fc2_w.astype(jnp.bfloat16), row2(fc2_b))

    return out.reshape(B, N, C)


# speedup vs baseline: 3.2438x; 1.1037x over previous
import jax
import jax.numpy as jnp
import numpy as np
from jax.experimental import pallas as pl

B, N, C, H = 2, 2048, 768, 12
D = C // H
HID = 4 * C
EPS = 1e-5
SCALE = D ** -0.5

ROWS = B * N
_QSCALE = np.concatenate([np.full((C,), SCALE, np.float32),
                          np.ones((2 * C,), np.float32)])
RBLK = 1024
NQ = N // RBLK


def _layernorm(xf, g, b):
    mu = jnp.mean(xf, axis=-1, keepdims=True)
    xc = xf - mu
    var = jnp.mean(xc * xc, axis=-1, keepdims=True)
    return xc * jax.lax.rsqrt(var + EPS) * g + b


def _qkv_kernel(x_ref, g_ref, b_ref, w_ref, bias_ref, out_ref):
    for sub in range(2):
        rows = pl.ds(sub * (RBLK // 2), RBLK // 2)
        h = _layernorm(x_ref[rows, :], g_ref[...],
                       b_ref[...]).astype(jnp.bfloat16)
        acc = jax.lax.dot_general(
            h, w_ref[...], (((1,), (0,)), ((), ())),
            preferred_element_type=jnp.float32)
        out_ref[rows, :] = (acc + bias_ref[...]).astype(jnp.bfloat16)


def _attn_kernel(q_ref, k_ref, v_ref, o_ref):
    outs = []
    for j in range(2):
        sl = pl.ds(j * D, D)
        q = q_ref[:, sl]
        k = k_ref[:, sl]
        v = v_ref[:, sl]
        s = jax.lax.dot_general(
            q, k, (((1,), (1,)), ((), ())),
            preferred_element_type=jnp.float32)
        p = jnp.exp(s.astype(jnp.bfloat16))
        v_aug = jnp.concatenate(
            [v, jnp.ones((N, D), jnp.bfloat16)], axis=1)
        o_aug = jax.lax.dot_general(
            p, v_aug, (((1,), (0,)), ((), ())),
            preferred_element_type=jnp.float32)
        outs.append((o_aug[:, :D] / o_aug[:, D:D + 1]).astype(jnp.bfloat16))
    o_ref[...] = jnp.concatenate(outs, axis=1)


def _mlp_kernel(o_ref, x_ref, pw_ref, pb_ref, g2_ref, b2_ref,
                w1_ref, b1_ref, w2_ref, b2b_ref, out_ref):
    for sub in range(2):
        rows = pl.ds(sub * (RBLK // 2), RBLK // 2)
        proj = jax.lax.dot_general(
            o_ref[rows, :], pw_ref[...], (((1,), (0,)), ((), ())),
            preferred_element_type=jnp.float32)
        x1 = proj + pb_ref[...] + x_ref[rows, :]
        h = _layernorm(x1, g2_ref[...], b2_ref[...]).astype(jnp.bfloat16)
        acc = b2b_ref[...] + x1
        for c in range(4):
            cols = pl.ds(c * (HID // 4), HID // 4)
            h1 = jax.lax.dot_general(
                h, w1_ref[:, cols], (((1,), (0,)), ((), ())),
                preferred_element_type=jnp.float32) + b1_ref[:, cols]
            g = 0.5 * h1 * (1.0 + jax.lax.erf(h1 * (2.0 ** -0.5)))
            acc = acc + jax.lax.dot_general(
                g.astype(jnp.bfloat16), w2_ref[cols, :],
                (((1,), (0,)), ((), ())),
                preferred_element_type=jnp.float32)
        out_ref[rows, :] = acc


@jax.jit
def kernel(x, norm1_g, norm1_b, qkv_w, qkv_b, proj_w, proj_b,
           norm2_g, norm2_b, fc1_w, fc1_b, fc2_w, fc2_b):
    xf = x.reshape(ROWS, C)
    row2 = lambda a: a.reshape(1, -1)

    qkv = pl.pallas_call(
        _qkv_kernel,
        grid=(ROWS // RBLK,),
        in_specs=[
            pl.BlockSpec((RBLK, C), lambda i: (i, 0)),
            pl.BlockSpec((1, C), lambda i: (0, 0)),
            pl.BlockSpec((1, C), lambda i: (0, 0)),
            pl.BlockSpec((C, 3 * C), lambda i: (0, 0)),
            pl.BlockSpec((1, 3 * C), lambda i: (0, 0)),
        ],
        out_specs=pl.BlockSpec((RBLK, 3 * C), lambda i: (i, 0)),
        out_shape=jax.ShapeDtypeStruct((ROWS, 3 * C), jnp.bfloat16),
    )(xf, row2(norm1_g), row2(norm1_b),
      (qkv_w * _QSCALE).astype(jnp.bfloat16), row2(qkv_b * _QSCALE))

    attn_out = pl.pallas_call(
        _attn_kernel,
        grid=(B, H // 2, NQ),
        in_specs=[
            pl.BlockSpec((RBLK, 2 * D), lambda b, h, i: (b * NQ + i, h)),
            pl.BlockSpec((N, 2 * D), lambda b, h, i: (b, H // 2 + h)),
            pl.BlockSpec((N, 2 * D), lambda b, h, i: (b, H + h)),
        ],
        out_specs=pl.BlockSpec((RBLK, 2 * D), lambda b, h, i: (b * NQ + i, h)),
        out_shape=jax.ShapeDtypeStruct((ROWS, C), jnp.bfloat16),
    )(qkv, qkv, qkv)

    out = pl.pallas_call(
        _mlp_kernel,
        grid=(ROWS // RBLK,),
        in_specs=[
            pl.BlockSpec((RBLK, C), lambda i: (i, 0)),
            pl.BlockSpec((RBLK, C), lambda i: (i, 0)),
            pl.BlockSpec((C, C), lambda i: (0, 0)),
            pl.BlockSpec((1, C), lambda i: (0, 0)),
            pl.BlockSpec((1, C), lambda i: (0, 0)),
            pl.BlockSpec((1, C), lambda i: (0, 0)),
            pl.BlockSpec((C, HID), lambda i: (0, 0)),
            pl.BlockSpec((1, HID), lambda i: (0, 0)),
            pl.BlockSpec((HID, C), lambda i: (0, 0)),
            pl.BlockSpec((1, C), lambda i: (0, 0)),
        ],
        out_specs=pl.BlockSpec((RBLK, C), lambda i: (i, 0)),
        out_shape=jax.ShapeDtypeStruct((ROWS, C), jnp.float32),
    )(attn_out, xf, proj_w.astype(jnp.bfloat16), row2(proj_b),
      row2(norm2_g), row2(norm2_b),
      fc1_w.astype(jnp.bfloat16), row2(fc1_b),
      fc2_w.astype(jnp.bfloat16), row2(fc2_b))

    return out.reshape(B, N, C)


# speedup vs baseline: 3.3328x; 1.0274x over previous
import jax
import jax.numpy as jnp
import numpy as np
from jax.experimental import pallas as pl

B, N, C, H = 2, 2048, 768, 12
D = C // H
HID = 4 * C
EPS = 1e-5
SCALE = D ** -0.5

ROWS = B * N
_QSCALE = np.concatenate([np.full((C,), SCALE, np.float32),
                          np.ones((2 * C,), np.float32)])
RBLK = 1024
ABLK = 2048
NQ = N // ABLK


def _layernorm(xf, g, b):
    mu = jnp.mean(xf, axis=-1, keepdims=True)
    xc = xf - mu
    var = jnp.mean(xc * xc, axis=-1, keepdims=True)
    return xc * jax.lax.rsqrt(var + EPS) * g + b


def _qkv_kernel(x_ref, g_ref, b_ref, w_ref, bias_ref, out_ref):
    for sub in range(2):
        rows = pl.ds(sub * (RBLK // 2), RBLK // 2)
        h = _layernorm(x_ref[rows, :], g_ref[...],
                       b_ref[...]).astype(jnp.bfloat16)
        acc = jax.lax.dot_general(
            h, w_ref[...], (((1,), (0,)), ((), ())),
            preferred_element_type=jnp.float32)
        out_ref[rows, :] = (acc + bias_ref[...]).astype(jnp.bfloat16)


def _attn_kernel(q_ref, k_ref, v_ref, o_ref):
    outs = []
    for j in range(2):
        sl = pl.ds(j * D, D)
        q = q_ref[:, sl]
        k = k_ref[:, sl]
        v = v_ref[:, sl]
        s = jax.lax.dot_general(
            q, k, (((1,), (1,)), ((), ())),
            preferred_element_type=jnp.float32)
        p = jnp.exp(s.astype(jnp.bfloat16))
        v_aug = jnp.concatenate(
            [v, jnp.ones((N, D), jnp.bfloat16)], axis=1)
        o_aug = jax.lax.dot_general(
            p, v_aug, (((1,), (0,)), ((), ())),
            preferred_element_type=jnp.float32)
        outs.append((o_aug[:, :D] / o_aug[:, D:D + 1]).astype(jnp.bfloat16))
    o_ref[...] = jnp.concatenate(outs, axis=1)


def _mlp_kernel(o_ref, x_ref, pw_ref, pb_ref, g2_ref, b2_ref,
                w1_ref, b1_ref, w2_ref, b2b_ref, out_ref):
    for sub in range(2):
        rows = pl.ds(sub * (RBLK // 2), RBLK // 2)
        proj = jax.lax.dot_general(
            o_ref[rows, :], pw_ref[...], (((1,), (0,)), ((), ())),
            preferred_element_type=jnp.float32)
        x1 = proj + pb_ref[...] + x_ref[rows, :]
        h = _layernorm(x1, g2_ref[...], b2_ref[...]).astype(jnp.bfloat16)
        acc = b2b_ref[...] + x1
        for c in range(4):
            cols = pl.ds(c * (HID // 4), HID // 4)
            h1 = jax.lax.dot_general(
                h, w1_ref[:, cols], (((1,), (0,)), ((), ())),
                preferred_element_type=jnp.float32) + b1_ref[:, cols]
            g = 0.5 * h1 * (1.0 + jax.lax.erf(h1 * (2.0 ** -0.5)))
            acc = acc + jax.lax.dot_general(
                g.astype(jnp.bfloat16), w2_ref[cols, :],
                (((1,), (0,)), ((), ())),
                preferred_element_type=jnp.float32)
        out_ref[rows, :] = acc


@jax.jit
def kernel(x, norm1_g, norm1_b, qkv_w, qkv_b, proj_w, proj_b,
           norm2_g, norm2_b, fc1_w, fc1_b, fc2_w, fc2_b):
    xf = x.reshape(ROWS, C)
    row2 = lambda a: a.reshape(1, -1)

    qkv = pl.pallas_call(
        _qkv_kernel,
        grid=(ROWS // RBLK,),
        in_specs=[
            pl.BlockSpec((RBLK, C), lambda i: (i, 0)),
            pl.BlockSpec((1, C), lambda i: (0, 0)),
            pl.BlockSpec((1, C), lambda i: (0, 0)),
            pl.BlockSpec((C, 3 * C), lambda i: (0, 0)),
            pl.BlockSpec((1, 3 * C), lambda i: (0, 0)),
        ],
        out_specs=pl.BlockSpec((RBLK, 3 * C), lambda i: (i, 0)),
        out_shape=jax.ShapeDtypeStruct((ROWS, 3 * C), jnp.bfloat16),
    )(xf, row2(norm1_g), row2(norm1_b),
      (qkv_w * _QSCALE).astype(jnp.bfloat16), row2(qkv_b * _QSCALE))

    attn_out = pl.pallas_call(
        _attn_kernel,
        grid=(B, H // 2, NQ),
        in_specs=[
            pl.BlockSpec((ABLK, 2 * D), lambda b, h, i: (b * NQ + i, h)),
            pl.BlockSpec((N, 2 * D), lambda b, h, i: (b, H // 2 + h)),
            pl.BlockSpec((N, 2 * D), lambda b, h, i: (b, H + h)),
        ],
        out_specs=pl.BlockSpec((ABLK, 2 * D), lambda b, h, i: (b * NQ + i, h)),
        out_shape=jax.ShapeDtypeStruct((ROWS, C), jnp.bfloat16),
    )(qkv, qkv, qkv)

    out = pl.pallas_call(
        _mlp_kernel,
        grid=(ROWS // RBLK,),
        in_specs=[
            pl.BlockSpec((RBLK, C), lambda i: (i, 0)),
            pl.BlockSpec((RBLK, C), lambda i: (i, 0)),
            pl.BlockSpec((C, C), lambda i: (0, 0)),
            pl.BlockSpec((1, C), lambda i: (0, 0)),
            pl.BlockSpec((1, C), lambda i: (0, 0)),
            pl.BlockSpec((1, C), lambda i: (0, 0)),
            pl.BlockSpec((C, HID), lambda i: (0, 0)),
            pl.BlockSpec((1, HID), lambda i: (0, 0)),
            pl.BlockSpec((HID, C), lambda i: (0, 0)),
            pl.BlockSpec((1, C), lambda i: (0, 0)),
        ],
        out_specs=pl.BlockSpec((RBLK, C), lambda i: (i, 0)),
        out_shape=jax.ShapeDtypeStruct((ROWS, C), jnp.float32),
    )(attn_out, xf, proj_w.astype(jnp.bfloat16), row2(proj_b),
      row2(norm2_g), row2(norm2_b),
      fc1_w.astype(jnp.bfloat16), row2(fc1_b),
      fc2_w.astype(jnp.bfloat16), row2(fc2_b))

    return out.reshape(B, N, C)
